# Initial kernel scaffold; baseline (speedup 1.0000x reference)
import jax
import jax.numpy as jnp
from jax.experimental import pallas as pl


def kernel(node_features, edges, edge_weights, input_node_indices, params):
    raise NotImplementedError("write your pallas kernel here")


# speedup vs baseline: 3.8457x; 3.8457x over previous
import functools
import numpy as np
import jax
import jax.numpy as jnp
from jax import lax
from jax.experimental import pallas as pl
from jax.experimental.pallas import tpu as pltpu
from jax.experimental.pallas import tpu_sc as plsc

N = 100000
E = 1600000
NPRED = 10000
NCLS = 40
HID = 32
BN_EPS = 1e-3

NSUB = 16
CH = 128
E_PAD = 1601536
EPS_E = E_PAD // NSUB
NCHUNK = EPS_E // CH
N_ACC = 100096
ZS = N_ACC // NSUB
NP_PAD = 12288

BT = 1000
NBLK = N // BT


def _gelu(x):
    return x * 0.5 * (1.0 + lax.erf(x * np.float32(1.0 / np.sqrt(2.0))))


def _fold_ffn(p, pre):
    s1 = p[pre + '_bn1_g'] * np.float32(1.0 / np.sqrt(1.0 + BN_EPS))
    w1 = s1[:, None] * p[pre + '_d1_w']
    b1 = p[pre + '_bn1_b'] @ p[pre + '_d1_w'] + p[pre + '_d1_b']
    s2 = p[pre + '_bn2_g'] * np.float32(1.0 / np.sqrt(1.0 + BN_EPS))
    w2 = s2[:, None] * p[pre + '_d2_w']
    b2 = p[pre + '_bn2_b'] @ p[pre + '_d2_w'] + p[pre + '_d2_b']
    return w1, b1[None, :], w2, b2[None, :]


def _conv_mats(k, b):
    terms = []
    for qi in range(2):
        for qj in range(2):
            w = 0.0
            for a in range(3):
                for bb in range(3):
                    sel = np.zeros((128, 32), np.float32)
                    for pi in range(8):
                        for pj in range(4):
                            ii = 2 * pi + qi + a - 1
                            jj = 2 * pj + qj + bb - 1
                            if 0 <= ii < 16 and 0 <= jj < 8:
                                sel[ii * 8 + jj, pi * 4 + pj] = 1.0
                    w = w + sel[:, :, None] * k[a, bb, 0][None, None, :]
            terms.append(w.reshape(128, 128))
    wc = jnp.stack(terms)
    bvec = jnp.tile(b, 32)[None, :]
    return wc, bvec


def _full(spec_shape):
    return pl.BlockSpec(spec_shape, lambda i: tuple(0 for _ in spec_shape))


def _sum_body(ew_ref, out_ref):
    @pl.when(pl.program_id(0) == 0)
    def _():
        out_ref[0, 0] = 0.0
    out_ref[0, 0] += jnp.sum(ew_ref[...])


def _encode_body(nf_ref, wc_ref, bc_ref, pw1, pb1, pw2, pb2,
                 mw1, mb1, mw2, mb2, x0_ref, msg_ref):
    nf = nf_ref[...]
    y = jnp.maximum(jnp.dot(nf, wc_ref[0], preferred_element_type=jnp.float32)
                    + bc_ref[...], 0.0)
    for q in range(1, 4):
        t = jnp.maximum(jnp.dot(nf, wc_ref[q], preferred_element_type=jnp.float32)
                        + bc_ref[...], 0.0)
        y = jnp.maximum(y, t)
    x = _gelu(jnp.dot(y, pw1[...], preferred_element_type=jnp.float32) + pb1[...])
    x = _gelu(jnp.dot(x, pw2[...], preferred_element_type=jnp.float32) + pb2[...])
    m = _gelu(jnp.dot(x, mw1[...], preferred_element_type=jnp.float32) + mb1[...])
    m = _gelu(jnp.dot(m, mw2[...], preferred_element_type=jnp.float32) + mb2[...])
    x0_ref[...] = x
    msg_ref[0] = m[:, :16]
    msg_ref[1] = m[:, 16:]


def _update_body(x_ref, agg_ref, s_ref, uw1a, uw1b0, uw1b1, ub1, uw2, ub2,
                 mw1, mb1, mw2, mb2, x1_ref, msg_ref):
    inv = 1.0 / s_ref[0, 0]
    x = x_ref[...]
    ga = jnp.dot(agg_ref[0], uw1b0[...], preferred_element_type=jnp.float32)
    gb = jnp.dot(agg_ref[1], uw1b1[...], preferred_element_type=jnp.float32)
    u = _gelu(jnp.dot(x, uw1a[...], preferred_element_type=jnp.float32)
              + (ga + gb) * inv + ub1[...])
    u = _gelu(jnp.dot(u, uw2[...], preferred_element_type=jnp.float32) + ub2[...])
    u = u * lax.rsqrt(jnp.maximum(jnp.sum(u * u, axis=1, keepdims=True), 1e-12))
    x1 = u + x
    m = _gelu(jnp.dot(x1, mw1[...], preferred_element_type=jnp.float32) + mb1[...])
    m = _gelu(jnp.dot(m, mw2[...], preferred_element_type=jnp.float32) + mb2[...])
    x1_ref[...] = x1
    msg_ref[0] = m[:, :16]
    msg_ref[1] = m[:, 16:]


def _final_body(x_ref, agg_ref, s_ref, uw1a, uw1b0, uw1b1, ub1, uw2, ub2,
                pw1, pb1, pw2, pb2, x3_ref):
    inv = 1.0 / s_ref[0, 0]
    x = x_ref[...]
    ga = jnp.dot(agg_ref[0], uw1b0[...], preferred_element_type=jnp.float32)
    gb = jnp.dot(agg_ref[1], uw1b1[...], preferred_element_type=jnp.float32)
    u = _gelu(jnp.dot(x, uw1a[...], preferred_element_type=jnp.float32)
              + (ga + gb) * inv + ub1[...])
    u = _gelu(jnp.dot(u, uw2[...], preferred_element_type=jnp.float32) + ub2[...])
    u = u * lax.rsqrt(jnp.maximum(jnp.sum(u * u, axis=1, keepdims=True), 1e-12))
    x2 = u + x
    t = _gelu(jnp.dot(x2, pw1[...], preferred_element_type=jnp.float32) + pb1[...])
    x3_ref[...] = _gelu(jnp.dot(t, pw2[...], preferred_element_type=jnp.float32)
                        + pb2[...])


def _logits_body(emb_ref, w_ref, b_ref, out_ref):
    out_ref[...] = (jnp.dot(emb_ref[...], w_ref[...],
                            preferred_element_type=jnp.float32) + b_ref[...])


def _sc_mesh():
    return plsc.VectorSubcoreMesh(core_axis_name="c", subcore_axis_name="s",
                                  num_cores=2, num_subcores=NSUB)


@functools.partial(
    pl.kernel,
    out_type=jax.ShapeDtypeStruct((2, N_ACC, 16), jnp.float32),
    mesh=_sc_mesh(),
    compiler_params=pltpu.CompilerParams(use_tc_tiling_on_sc=False),
    scratch_types=[
        pltpu.VMEM((CH,), jnp.int32),
        pltpu.VMEM((CH,), jnp.int32),
        pltpu.VMEM((CH,), jnp.float32),
        pltpu.VMEM((CH, 16), jnp.float32),
        pltpu.VMEM_SHARED((N_ACC, 16), jnp.float32),
        pltpu.SemaphoreType.DMA,
    ],
)
def _agg_kernel(msg_hbm, src_hbm, dst_hbm, ew_hbm, zero_hbm, out_hbm,
                srcv, dstv, eww, rows, shared, sem):
    c = lax.axis_index("c")
    s = lax.axis_index("s")
    pltpu.sync_copy(zero_hbm.at[pl.ds(s * ZS, ZS)], shared.at[pl.ds(s * ZS, ZS)])
    plsc.subcore_barrier()
    base = s * EPS_E
    coff = c * N

    def chunk(j, carry):
        b = base + j * CH
        pltpu.sync_copy(src_hbm.at[pl.ds(b, CH)], srcv)
        pltpu.sync_copy(dst_hbm.at[pl.ds(b, CH)], dstv)
        pltpu.sync_copy(ew_hbm.at[pl.ds(b, CH)], eww)
        for i in range(CH // 16):
            sl = pl.ds(i * 16, 16)
            srcv[sl] = srcv[sl] + coff
        pltpu.async_copy(msg_hbm.at[srcv], rows, sem).wait()

        def scale(k, carry2):
            wv = eww[pl.ds(k * 16, 16)]
            for e in range(16):
                r = k * 16 + e
                rows[r, :] = rows[r, :] * wv[e]
            return carry2
        lax.fori_loop(0, CH // 16, scale, 0)
        pltpu.sync_copy(rows, shared.at[dstv], add=True)
        return carry

    lax.fori_loop(0, NCHUNK, chunk, 0)
    plsc.subcore_barrier()
    pltpu.sync_copy(shared.at[pl.ds(s * ZS, ZS)],
                    out_hbm.at[c, pl.ds(s * ZS, ZS)])


@functools.partial(
    pl.kernel,
    out_type=jax.ShapeDtypeStruct((NP_PAD, HID), jnp.float32),
    mesh=_sc_mesh(),
    compiler_params=pltpu.CompilerParams(use_tc_tiling_on_sc=False),
    scratch_types=[
        pltpu.VMEM((CH,), jnp.int32),
        pltpu.VMEM((CH, HID), jnp.float32),
        pltpu.SemaphoreType.DMA,
    ],
)
def _gather_kernel(x_hbm, idx_hbm, out_hbm, idxv, rows, sem):
    c = lax.axis_index("c")
    s = lax.axis_index("s")
    wid = s * 2 + c
    for j in range(NP_PAD // (32 * CH)):
        b = wid * (NP_PAD // 32) + j * CH
        pltpu.sync_copy(idx_hbm.at[pl.ds(b, CH)], idxv)
        pltpu.async_copy(x_hbm.at[idxv], rows, sem).wait()
        pltpu.sync_copy(rows, out_hbm.at[pl.ds(b, CH)])


def kernel(node_features, edges, edge_weights, input_node_indices, params):
    p = params
    f32 = jnp.float32
    nf = node_features.reshape(N, 128)
    pad_e = E_PAD - E
    srcp = jnp.pad(edges[1].astype(jnp.int32), (0, pad_e))
    dstp = jnp.pad(edges[0].astype(jnp.int32), (0, pad_e))
    ewp = jnp.pad(edge_weights, (0, pad_e))
    idxp = jnp.pad(input_node_indices.astype(jnp.int32), (0, NP_PAD - NPRED))
    zeros_h = jnp.zeros((N_ACC, 16), f32)

    wc, bc = _conv_mats(p['conv_k'], p['conv_b'])
    pre = _fold_ffn(p, 'pre')
    c1p = _fold_ffn(p, 'c1_p')
    c1u = _fold_ffn(p, 'c1_u')
    c2p = _fold_ffn(p, 'c2_p')
    c2u = _fold_ffn(p, 'c2_u')
    post = _fold_ffn(p, 'post')

    def split_u(u):
        w1, b1, w2, b2 = u
        return (w1[:HID], w1[HID:HID + 16], w1[HID + 16:], b1, w2, b2)

    c1u = split_u(c1u)
    c2u = split_u(c2u)

    ew2 = ewp.reshape(12512, 128)
    ssum = pl.pallas_call(
        _sum_body,
        grid=(391,),
        in_specs=[pl.BlockSpec((32, 128), lambda i: (i, 0))],
        out_specs=pl.BlockSpec(memory_space=pltpu.SMEM),
        out_shape=jax.ShapeDtypeStruct((1, 1), f32),
    )(ew2)

    wspecs = [_full(w.shape) for w in (wc, bc, *pre, *c1p)]
    x0, msg1 = pl.pallas_call(
        _encode_body,
        grid=(NBLK,),
        in_specs=[pl.BlockSpec((BT, 128), lambda i: (i, 0))] + wspecs,
        out_specs=[pl.BlockSpec((BT, HID), lambda i: (i, 0)),
                   pl.BlockSpec((2, BT, 16), lambda i: (0, i, 0))],
        out_shape=[jax.ShapeDtypeStruct((N, HID), f32),
                   jax.ShapeDtypeStruct((2, N, 16), f32)],
    )(nf, wc, bc, *pre, *c1p)

    agg1 = _agg_kernel(msg1.reshape(2 * N, 16), srcp, dstp, ewp, zeros_h)

    def update(x, agg, uw, mw, body, n_out):
        ins = (x, agg, ssum, *uw, *mw)
        specs = [pl.BlockSpec((BT, HID), lambda i: (i, 0)),
                 pl.BlockSpec((2, BT, 16), lambda i: (0, i, 0)),
                 pl.BlockSpec(memory_space=pltpu.SMEM)]
        specs += [_full(w.shape) for w in (*uw, *mw)]
        if n_out == 2:
            outs = [pl.BlockSpec((BT, HID), lambda i: (i, 0)),
                    pl.BlockSpec((2, BT, 16), lambda i: (0, i, 0))]
            oshape = [jax.ShapeDtypeStruct((N, HID), f32),
                      jax.ShapeDtypeStruct((2, N, 16), f32)]
        else:
            outs = pl.BlockSpec((BT, HID), lambda i: (i, 0))
            oshape = jax.ShapeDtypeStruct((N, HID), f32)
        return pl.pallas_call(body, grid=(NBLK,), in_specs=specs,
                              out_specs=outs, out_shape=oshape)(*ins)

    x1, msg2 = update(x0, agg1, c1u, c2p, _update_body, 2)
    agg2 = _agg_kernel(msg2.reshape(2 * N, 16), srcp, dstp, ewp, zeros_h)
    x3 = update(x1, agg2, c2u, post, _final_body, 1)

    emb = _gather_kernel(x3, idxp)

    logits = pl.pallas_call(
        _logits_body,
        grid=(NP_PAD // 1024,),
        in_specs=[pl.BlockSpec((1024, HID), lambda i: (i, 0)),
                  _full((HID, NCLS)), _full((1, NCLS))],
        out_specs=pl.BlockSpec((1024, NCLS), lambda i: (i, 0)),
        out_shape=jax.ShapeDtypeStruct((NP_PAD, NCLS), f32),
    )(emb, p['log_w'], p['log_b'][None, :])

    return logits[:NPRED]


# speedup vs baseline: 4.8719x; 1.2669x over previous
import functools
import numpy as np
import jax
import jax.numpy as jnp
from jax import lax
from jax.experimental import pallas as pl
from jax.experimental.pallas import tpu as pltpu
from jax.experimental.pallas import tpu_sc as plsc

N = 100000
E = 1600000
NPRED = 10000
NCLS = 40
HID = 32
BN_EPS = 1e-3

NSUB = 16
CH = 128
CB = 512
E_PAD = 1605632
EPS_E = E_PAD // NSUB
NCHUNK = EPS_E // CB
N_ACC = 100096
ZS = N_ACC // NSUB
NP_PAD = 12288

BT = 1000
NBLK = N // BT


def _gelu(x):
    return x * 0.5 * (1.0 + lax.erf(x * np.float32(1.0 / np.sqrt(2.0))))


def _fold_ffn(p, pre):
    s1 = p[pre + '_bn1_g'] * np.float32(1.0 / np.sqrt(1.0 + BN_EPS))
    w1 = s1[:, None] * p[pre + '_d1_w']
    b1 = p[pre + '_bn1_b'] @ p[pre + '_d1_w'] + p[pre + '_d1_b']
    s2 = p[pre + '_bn2_g'] * np.float32(1.0 / np.sqrt(1.0 + BN_EPS))
    w2 = s2[:, None] * p[pre + '_d2_w']
    b2 = p[pre + '_bn2_b'] @ p[pre + '_d2_w'] + p[pre + '_d2_b']
    return w1, b1[None, :], w2, b2[None, :]


def _conv_mats(k, b):
    terms = []
    for qi in range(2):
        for qj in range(2):
            w = 0.0
            for a in range(3):
                for bb in range(3):
                    sel = np.zeros((128, 32), np.float32)
                    for pi in range(8):
                        for pj in range(4):
                            ii = 2 * pi + qi + a - 1
                            jj = 2 * pj + qj + bb - 1
                            if 0 <= ii < 16 and 0 <= jj < 8:
                                sel[ii * 8 + jj, pi * 4 + pj] = 1.0
                    w = w + sel[:, :, None] * k[a, bb, 0][None, None, :]
            terms.append(w.reshape(128, 128))
    wc = jnp.stack(terms)
    bvec = jnp.tile(b, 32)[None, :]
    return wc, bvec


def _full(spec_shape):
    return pl.BlockSpec(spec_shape, lambda i: tuple(0 for _ in spec_shape))


def _sum_body(ew_ref, out_ref):
    @pl.when(pl.program_id(0) == 0)
    def _():
        out_ref[0, 0] = 0.0
    out_ref[0, 0] += jnp.sum(ew_ref[...])


def _encode_body(nf_ref, wc_ref, bc_ref, pw1, pb1, pw2, pb2,
                 mw1, mb1, mw2, mb2, x0_ref, msg_ref):
    nf = nf_ref[...]
    y = jnp.maximum(jnp.dot(nf, wc_ref[0], preferred_element_type=jnp.float32,
                precision=lax.Precision.HIGHEST)
                    + bc_ref[...], 0.0)
    for q in range(1, 4):
        t = jnp.maximum(jnp.dot(nf, wc_ref[q], preferred_element_type=jnp.float32,
                precision=lax.Precision.HIGHEST)
                        + bc_ref[...], 0.0)
        y = jnp.maximum(y, t)
    x = _gelu(jnp.dot(y, pw1[...], preferred_element_type=jnp.float32,
                precision=lax.Precision.HIGHEST) + pb1[...])
    x = _gelu(jnp.dot(x, pw2[...], preferred_element_type=jnp.float32,
                precision=lax.Precision.HIGHEST) + pb2[...])
    m = _gelu(jnp.dot(x, mw1[...], preferred_element_type=jnp.float32,
                precision=lax.Precision.HIGHEST) + mb1[...])
    m = _gelu(jnp.dot(m, mw2[...], preferred_element_type=jnp.float32,
                precision=lax.Precision.HIGHEST) + mb2[...])
    x0_ref[...] = x
    msg_ref[0] = m[:, :16]
    msg_ref[1] = m[:, 16:]


def _update_body(x_ref, agg_ref, s_ref, uw1a, uw1b0, uw1b1, ub1, uw2, ub2,
                 mw1, mb1, mw2, mb2, x1_ref, msg_ref):
    inv = 1.0 / s_ref[0, 0]
    x = x_ref[...]
    ga = jnp.dot(agg_ref[0], uw1b0[...], preferred_element_type=jnp.float32,
                precision=lax.Precision.HIGHEST)
    gb = jnp.dot(agg_ref[1], uw1b1[...], preferred_element_type=jnp.float32,
                precision=lax.Precision.HIGHEST)
    u = _gelu(jnp.dot(x, uw1a[...], preferred_element_type=jnp.float32,
                precision=lax.Precision.HIGHEST)
              + (ga + gb) * inv + ub1[...])
    u = _gelu(jnp.dot(u, uw2[...], preferred_element_type=jnp.float32,
                precision=lax.Precision.HIGHEST) + ub2[...])
    u = u * lax.rsqrt(jnp.maximum(jnp.sum(u * u, axis=1, keepdims=True), 1e-12))
    x1 = u + x
    m = _gelu(jnp.dot(x1, mw1[...], preferred_element_type=jnp.float32,
                precision=lax.Precision.HIGHEST) + mb1[...])
    m = _gelu(jnp.dot(m, mw2[...], preferred_element_type=jnp.float32,
                precision=lax.Precision.HIGHEST) + mb2[...])
    x1_ref[...] = x1
    msg_ref[0] = m[:, :16]
    msg_ref[1] = m[:, 16:]


def _final_body(x_ref, agg_ref, s_ref, uw1a, uw1b0, uw1b1, ub1, uw2, ub2,
                pw1, pb1, pw2, pb2, x3_ref):
    inv = 1.0 / s_ref[0, 0]
    x = x_ref[...]
    ga = jnp.dot(agg_ref[0], uw1b0[...], preferred_element_type=jnp.float32,
                precision=lax.Precision.HIGHEST)
    gb = jnp.dot(agg_ref[1], uw1b1[...], preferred_element_type=jnp.float32,
                precision=lax.Precision.HIGHEST)
    u = _gelu(jnp.dot(x, uw1a[...], preferred_element_type=jnp.float32,
                precision=lax.Precision.HIGHEST)
              + (ga + gb) * inv + ub1[...])
    u = _gelu(jnp.dot(u, uw2[...], preferred_element_type=jnp.float32,
                precision=lax.Precision.HIGHEST) + ub2[...])
    u = u * lax.rsqrt(jnp.maximum(jnp.sum(u * u, axis=1, keepdims=True), 1e-12))
    x2 = u + x
    t = _gelu(jnp.dot(x2, pw1[...], preferred_element_type=jnp.float32,
                precision=lax.Precision.HIGHEST) + pb1[...])
    x3_ref[...] = _gelu(jnp.dot(t, pw2[...], preferred_element_type=jnp.float32,
                precision=lax.Precision.HIGHEST)
                        + pb2[...])


def _logits_body(emb_ref, w_ref, b_ref, out_ref):
    out_ref[...] = (jnp.dot(emb_ref[...], w_ref[...],
                            preferred_element_type=jnp.float32,
                precision=lax.Precision.HIGHEST) + b_ref[...])


def _sc_mesh():
    return plsc.VectorSubcoreMesh(core_axis_name="c", subcore_axis_name="s",
                                  num_cores=2, num_subcores=NSUB)


@functools.partial(
    pl.kernel,
    out_type=jax.ShapeDtypeStruct((2, N_ACC, 16), jnp.float32),
    mesh=_sc_mesh(),
    compiler_params=pltpu.CompilerParams(use_tc_tiling_on_sc=False),
    scratch_types=[
        pltpu.VMEM((4, CH), jnp.int32),
        pltpu.VMEM((4, CH), jnp.int32),
        pltpu.VMEM((4, CH), jnp.int32),
        pltpu.VMEM((4, CH), jnp.int32),
        pltpu.VMEM((CB,), jnp.float32),
        pltpu.VMEM((CB,), jnp.float32),
        pltpu.VMEM((CB, 16), jnp.float32),
        pltpu.VMEM((CB, 16), jnp.float32),
        pltpu.VMEM_SHARED((N_ACC, 16), jnp.float32),
        pltpu.SemaphoreType.DMA,
        pltpu.SemaphoreType.DMA,
        pltpu.SemaphoreType.DMA,
        pltpu.SemaphoreType.DMA,
        pltpu.SemaphoreType.DMA,
        pltpu.SemaphoreType.DMA,
        pltpu.SemaphoreType.DMA,
        pltpu.SemaphoreType.DMA,
    ],
)
def _agg_kernel(msg_hbm, src_hbm, dst_hbm, ew_hbm, zero_hbm, out_hbm,
                srcv0, srcv1, dstv0, dstv1, eww0, eww1, rows0, rows1,
                shared, si0, si1, sd0, sd1, sg0, sg1, ss0, ss1):
    c = lax.axis_index("c")
    s = lax.axis_index("s")
    pltpu.sync_copy(zero_hbm.at[pl.ds(s * ZS, ZS)], shared.at[pl.ds(s * ZS, ZS)])
    plsc.subcore_barrier()
    ebase = s * EPS_E
    rbase = s * (EPS_E // CH)
    coff = c * N_ACC

    bufs = ((srcv0, dstv0, eww0, rows0, si0, sd0, sg0, ss0),
            (srcv1, dstv1, eww1, rows1, si1, sd1, sg1, ss1))

    def srcew_descs(b, g):
        srcv, _, eww, _, si, _, _, _ = bufs[b]
        return [pltpu.make_async_copy(src_hbm.at[pl.ds(rbase + g * 4, 4)], srcv, si),
                pltpu.make_async_copy(ew_hbm.at[pl.ds(ebase + g * CB, CB)], eww, si)]

    def dst_desc(b, g):
        _, dstv, _, _, _, sd, _, _ = bufs[b]
        return pltpu.make_async_copy(dst_hbm.at[pl.ds(rbase + g * 4, 4)], dstv, sd)

    def gather_descs(b):
        srcv, _, _, rows, _, _, sg, _ = bufs[b]
        return [pltpu.make_async_copy(msg_hbm.at[srcv.at[j]],
                                      rows.at[pl.ds(j * CH, CH)], sg)
                for j in range(4)]

    def scatter_descs(b):
        _, dstv, _, rows, _, _, _, ss = bufs[b]
        return [pltpu.make_async_copy(rows.at[pl.ds(j * CH, CH)],
                                      shared.at[dstv.at[j]], ss)
                for j in range(4)]

    def adjust(b):
        srcv = bufs[b][0]
        for j in range(4):
            for i in range(CH // 16):
                sl = pl.ds(i * 16, 16)
                srcv[j, sl] = srcv[j, sl] + coff

    def scale(b):
        eww, rows = bufs[b][2], bufs[b][3]

        @plsc.parallel_loop(0, CB, 16, unroll=2)
        def _(i):
            wv = eww[pl.ds(i, 16)]
            for e in range(16):
                rows[i + e, :] = rows[i + e, :] * wv[e]

    def issue_scatter(b):
        _, dstv, _, rows, _, _, _, ss = bufs[b]
        for j in range(4):
            pltpu.async_copy(rows.at[pl.ds(j * CH, CH)],
                             shared.at[dstv.at[j]], ss, add=True)

    for d in srcew_descs(0, 0):
        d.start()
    dst_desc(0, 0).start()
    for d in srcew_descs(0, 0):
        d.wait()
    adjust(0)
    for d in gather_descs(0):
        d.start()
    for d in srcew_descs(1, 1):
        d.start()

    def pair(k, carry):
        for p in (0, 1):
            g = 2 * k + p
            q = 1 - p
            for d in gather_descs(p):
                d.wait()
            @pl.when(g >= 1)
            def _():
                for d in scatter_descs(q):
                    d.wait()

            @pl.when(g + 1 < NCHUNK)
            def _():
                for d in srcew_descs(q, g + 1):
                    d.wait()
                adjust(q)
                for d in gather_descs(q):
                    d.start()
                dst_desc(q, g + 1).start()
            scale(p)
            dst_desc(p, g).wait()
            issue_scatter(p)
            @pl.when(g + 2 < NCHUNK)
            def _():
                for d in srcew_descs(p, g + 2):
                    d.start()
        return carry

    lax.fori_loop(0, NCHUNK // 2, pair, 0)
    for d in scatter_descs((NCHUNK - 1) % 2):
        d.wait()
    plsc.subcore_barrier()
    pltpu.sync_copy(shared.at[pl.ds(s * ZS, ZS)],
                    out_hbm.at[c, pl.ds(s * ZS, ZS)])


@functools.partial(
    pl.kernel,
    out_type=jax.ShapeDtypeStruct((NP_PAD, HID), jnp.float32),
    mesh=_sc_mesh(),
    compiler_params=pltpu.CompilerParams(use_tc_tiling_on_sc=False),
    scratch_types=[
        pltpu.VMEM((CH,), jnp.int32),
        pltpu.VMEM((CH, HID), jnp.float32),
        pltpu.SemaphoreType.DMA,
    ],
)
def _gather_kernel(x_hbm, idx_hbm, out_hbm, idxv, rows, sem):
    c = lax.axis_index("c")
    s = lax.axis_index("s")
    wid = s * 2 + c
    for j in range(NP_PAD // (32 * CH)):
        b = wid * (NP_PAD // 32) + j * CH
        pltpu.sync_copy(idx_hbm.at[pl.ds(b, CH)], idxv)
        pltpu.async_copy(x_hbm.at[idxv], rows, sem).wait()
        pltpu.sync_copy(rows, out_hbm.at[pl.ds(b, CH)])


def kernel(node_features, edges, edge_weights, input_node_indices, params):
    p = params
    f32 = jnp.float32
    nf = node_features.reshape(N, 128)
    pad_e = E_PAD - E
    srcp = jnp.pad(edges[1].astype(jnp.int32), (0, pad_e))
    dstp = jnp.pad(edges[0].astype(jnp.int32), (0, pad_e))
    ewp = jnp.pad(edge_weights, (0, pad_e))
    idxp = jnp.pad(input_node_indices.astype(jnp.int32), (0, NP_PAD - NPRED))
    zeros_h = jnp.zeros((N_ACC, 16), f32)

    def _msg_pad(m):
        return jnp.pad(m, ((0, 0), (0, N_ACC - N), (0, 0))).reshape(2 * N_ACC, 16)

    wc, bc = _conv_mats(p['conv_k'], p['conv_b'])
    pre = _fold_ffn(p, 'pre')
    c1p = _fold_ffn(p, 'c1_p')
    c1u = _fold_ffn(p, 'c1_u')
    c2p = _fold_ffn(p, 'c2_p')
    c2u = _fold_ffn(p, 'c2_u')
    post = _fold_ffn(p, 'post')

    def split_u(u):
        w1, b1, w2, b2 = u
        return (w1[:HID], w1[HID:HID + 16], w1[HID + 16:], b1, w2, b2)

    c1u = split_u(c1u)
    c2u = split_u(c2u)

    ew2 = ewp.reshape(E_PAD // CH, CH)
    ssum = pl.pallas_call(
        _sum_body,
        grid=(E_PAD // CH // 32,),
        in_specs=[pl.BlockSpec((32, 128), lambda i: (i, 0))],
        out_specs=pl.BlockSpec(memory_space=pltpu.SMEM),
        out_shape=jax.ShapeDtypeStruct((1, 1), f32),
    )(ew2)

    wspecs = [_full(w.shape) for w in (wc, bc, *pre, *c1p)]
    x0, msg1 = pl.pallas_call(
        _encode_body,
        grid=(NBLK,),
        in_specs=[pl.BlockSpec((BT, 128), lambda i: (i, 0))] + wspecs,
        out_specs=[pl.BlockSpec((BT, HID), lambda i: (i, 0)),
                   pl.BlockSpec((2, BT, 16), lambda i: (0, i, 0))],
        out_shape=[jax.ShapeDtypeStruct((N, HID), f32),
                   jax.ShapeDtypeStruct((2, N, 16), f32)],
    )(nf, wc, bc, *pre, *c1p)

    src2d = srcp.reshape(E_PAD // CH, CH)
    dst2d = dstp.reshape(E_PAD // CH, CH)
    agg1 = _agg_kernel(_msg_pad(msg1), src2d, dst2d, ewp, zeros_h)

    def update(x, agg, uw, mw, body, n_out):
        ins = (x, agg, ssum, *uw, *mw)
        specs = [pl.BlockSpec((BT, HID), lambda i: (i, 0)),
                 pl.BlockSpec((2, BT, 16), lambda i: (0, i, 0)),
                 pl.BlockSpec(memory_space=pltpu.SMEM)]
        specs += [_full(w.shape) for w in (*uw, *mw)]
        if n_out == 2:
            outs = [pl.BlockSpec((BT, HID), lambda i: (i, 0)),
                    pl.BlockSpec((2, BT, 16), lambda i: (0, i, 0))]
            oshape = [jax.ShapeDtypeStruct((N, HID), f32),
                      jax.ShapeDtypeStruct((2, N, 16), f32)]
        else:
            outs = pl.BlockSpec((BT, HID), lambda i: (i, 0))
            oshape = jax.ShapeDtypeStruct((N, HID), f32)
        return pl.pallas_call(body, grid=(NBLK,), in_specs=specs,
                              out_specs=outs, out_shape=oshape)(*ins)

    x1, msg2 = update(x0, agg1, c1u, c2p, _update_body, 2)
    agg2 = _agg_kernel(_msg_pad(msg2), src2d, dst2d, ewp, zeros_h)
    x3 = update(x1, agg2, c2u, post, _final_body, 1)

    emb = _gather_kernel(x3, idxp)

    logits = pl.pallas_call(
        _logits_body,
        grid=(NP_PAD // 1024,),
        in_specs=[pl.BlockSpec((1024, HID), lambda i: (i, 0)),
                  _full((HID, NCLS)), _full((1, NCLS))],
        out_specs=pl.BlockSpec((1024, NCLS), lambda i: (i, 0)),
        out_shape=jax.ShapeDtypeStruct((NP_PAD, NCLS), f32),
    )(emb, p['log_w'], p['log_b'][None, :])

    return logits[:NPRED]


# speedup vs baseline: 11.6844x; 2.3983x over previous
import functools
import numpy as np
import jax
import jax.numpy as jnp
from jax import lax
from jax.experimental import pallas as pl
from jax.experimental.pallas import tpu as pltpu
from jax.experimental.pallas import tpu_sc as plsc

N = 100000
E = 1600000
NPRED = 10000
NCLS = 40
HID = 32
BN_EPS = 1e-3

NSUB = 16
CH = 128
CB = 512
E_PAD = 1605632
EPS_E = E_PAD // NSUB
NCHUNK = EPS_E // CB
N_ACC = 100096
ZS = N_ACC // NSUB
NP_PAD = 12288

BT = 1000
NBLK = N // BT


def _gelu(x):
    return x * 0.5 * (1.0 + lax.erf(x * np.float32(1.0 / np.sqrt(2.0))))


def _fold_ffn(p, pre):
    s1 = p[pre + '_bn1_g'] * np.float32(1.0 / np.sqrt(1.0 + BN_EPS))
    w1 = s1[:, None] * p[pre + '_d1_w']
    b1 = p[pre + '_bn1_b'] @ p[pre + '_d1_w'] + p[pre + '_d1_b']
    s2 = p[pre + '_bn2_g'] * np.float32(1.0 / np.sqrt(1.0 + BN_EPS))
    w2 = s2[:, None] * p[pre + '_d2_w']
    b2 = p[pre + '_bn2_b'] @ p[pre + '_d2_w'] + p[pre + '_d2_b']
    return w1, b1[None, :], w2, b2[None, :]


def _conv_mats(k, b):
    terms = []
    for qi in range(2):
        for qj in range(2):
            w = 0.0
            for a in range(3):
                for bb in range(3):
                    sel = np.zeros((128, 32), np.float32)
                    for pi in range(8):
                        for pj in range(4):
                            ii = 2 * pi + qi + a - 1
                            jj = 2 * pj + qj + bb - 1
                            if 0 <= ii < 16 and 0 <= jj < 8:
                                sel[ii * 8 + jj, pi * 4 + pj] = 1.0
                    w = w + sel[:, :, None] * k[a, bb, 0][None, None, :]
            terms.append(w.reshape(128, 128))
    wc = jnp.stack(terms)
    bvec = jnp.tile(b, 32)[None, :]
    return wc, bvec


def _full(spec_shape):
    return pl.BlockSpec(spec_shape, lambda i: tuple(0 for _ in spec_shape))


def _sum_body(ew_ref, out_ref):
    @pl.when(pl.program_id(0) == 0)
    def _():
        out_ref[0, 0] = 0.0
    out_ref[0, 0] += jnp.sum(ew_ref[...])


def _encode_body(nf_ref, wc_ref, bc_ref, pw1, pb1, pw2, pb2,
                 mw1, mb1, mw2, mb2, x0_ref, msg_ref):
    nf = nf_ref[...]
    y = jnp.maximum(jnp.dot(nf, wc_ref[0], preferred_element_type=jnp.float32)
                    + bc_ref[...], 0.0)
    for q in range(1, 4):
        t = jnp.maximum(jnp.dot(nf, wc_ref[q], preferred_element_type=jnp.float32)
                        + bc_ref[...], 0.0)
        y = jnp.maximum(y, t)
    x = _gelu(jnp.dot(y, pw1[...], preferred_element_type=jnp.float32) + pb1[...])
    x = _gelu(jnp.dot(x, pw2[...], preferred_element_type=jnp.float32) + pb2[...])
    m = _gelu(jnp.dot(x, mw1[...], preferred_element_type=jnp.float32) + mb1[...])
    m = _gelu(jnp.dot(m, mw2[...], preferred_element_type=jnp.float32) + mb2[...])
    x0_ref[...] = x
    msg_ref[0] = m[:, :16]
    msg_ref[1] = m[:, 16:]


def _update_body(x_ref, agg_ref, s_ref, uw1a, uw1b0, uw1b1, ub1, uw2, ub2,
                 mw1, mb1, mw2, mb2, x1_ref, msg_ref):
    inv = 1.0 / s_ref[0, 0]
    x = x_ref[...]
    ga = jnp.dot(agg_ref[0], uw1b0[...], preferred_element_type=jnp.float32)
    gb = jnp.dot(agg_ref[1], uw1b1[...], preferred_element_type=jnp.float32)
    u = _gelu(jnp.dot(x, uw1a[...], preferred_element_type=jnp.float32)
              + (ga + gb) * inv + ub1[...])
    u = _gelu(jnp.dot(u, uw2[...], preferred_element_type=jnp.float32) + ub2[...])
    u = u * lax.rsqrt(jnp.maximum(jnp.sum(u * u, axis=1, keepdims=True), 1e-12))
    x1 = u + x
    m = _gelu(jnp.dot(x1, mw1[...], preferred_element_type=jnp.float32) + mb1[...])
    m = _gelu(jnp.dot(m, mw2[...], preferred_element_type=jnp.float32) + mb2[...])
    x1_ref[...] = x1
    msg_ref[0] = m[:, :16]
    msg_ref[1] = m[:, 16:]


def _final_body(x_ref, agg_ref, s_ref, uw1a, uw1b0, uw1b1, ub1, uw2, ub2,
                pw1, pb1, pw2, pb2, x3_ref):
    inv = 1.0 / s_ref[0, 0]
    x = x_ref[...]
    ga = jnp.dot(agg_ref[0], uw1b0[...], preferred_element_type=jnp.float32)
    gb = jnp.dot(agg_ref[1], uw1b1[...], preferred_element_type=jnp.float32)
    u = _gelu(jnp.dot(x, uw1a[...], preferred_element_type=jnp.float32)
              + (ga + gb) * inv + ub1[...])
    u = _gelu(jnp.dot(u, uw2[...], preferred_element_type=jnp.float32) + ub2[...])
    u = u * lax.rsqrt(jnp.maximum(jnp.sum(u * u, axis=1, keepdims=True), 1e-12))
    x2 = u + x
    t = _gelu(jnp.dot(x2, pw1[...], preferred_element_type=jnp.float32) + pb1[...])
    x3_ref[...] = _gelu(jnp.dot(t, pw2[...], preferred_element_type=jnp.float32)
                        + pb2[...])


def _logits_body(emb_ref, w_ref, b_ref, out_ref):
    out_ref[...] = (jnp.dot(emb_ref[...], w_ref[...],
                            preferred_element_type=jnp.float32) + b_ref[...])


def _sc_mesh():
    return plsc.VectorSubcoreMesh(core_axis_name="c", subcore_axis_name="s",
                                  num_cores=2, num_subcores=NSUB)


@functools.partial(
    pl.kernel,
    out_type=jax.ShapeDtypeStruct((2, N_ACC, 16), jnp.float32),
    mesh=_sc_mesh(),
    compiler_params=pltpu.CompilerParams(use_tc_tiling_on_sc=False),
    scratch_types=[
        pltpu.VMEM((4, CH), jnp.int32),
        pltpu.VMEM((4, CH), jnp.int32),
        pltpu.VMEM((4, CH), jnp.int32),
        pltpu.VMEM((4, CH), jnp.int32),
        pltpu.VMEM((CB,), jnp.float32),
        pltpu.VMEM((CB,), jnp.float32),
        pltpu.VMEM((CB, 16), jnp.float32),
        pltpu.VMEM((CB, 16), jnp.float32),
        pltpu.VMEM_SHARED((N_ACC, 16), jnp.float32),
        pltpu.SemaphoreType.DMA,
        pltpu.SemaphoreType.DMA,
        pltpu.SemaphoreType.DMA,
        pltpu.SemaphoreType.DMA,
        pltpu.SemaphoreType.DMA,
        pltpu.SemaphoreType.DMA,
        pltpu.SemaphoreType.DMA,
        pltpu.SemaphoreType.DMA,
    ],
)
def _agg_kernel(msg_hbm, src_hbm, dst_hbm, ew_hbm, zero_hbm, out_hbm,
                srcv0, srcv1, dstv0, dstv1, eww0, eww1, rows0, rows1,
                shared, si0, si1, sd0, sd1, sg0, sg1, ss0, ss1):
    c = lax.axis_index("c")
    s = lax.axis_index("s")
    pltpu.sync_copy(zero_hbm.at[pl.ds(s * ZS, ZS)], shared.at[pl.ds(s * ZS, ZS)])
    plsc.subcore_barrier()
    ebase = s * EPS_E
    rbase = s * (EPS_E // CH)
    coff = c * N_ACC

    bufs = ((srcv0, dstv0, eww0, rows0, si0, sd0, sg0, ss0),
            (srcv1, dstv1, eww1, rows1, si1, sd1, sg1, ss1))

    def srcew_descs(b, g):
        srcv, _, eww, _, si, _, _, _ = bufs[b]
        return [pltpu.make_async_copy(src_hbm.at[pl.ds(rbase + g * 4, 4)], srcv, si),
                pltpu.make_async_copy(ew_hbm.at[pl.ds(ebase + g * CB, CB)], eww, si)]

    def dst_desc(b, g):
        _, dstv, _, _, _, sd, _, _ = bufs[b]
        return pltpu.make_async_copy(dst_hbm.at[pl.ds(rbase + g * 4, 4)], dstv, sd)

    def gather_descs(b):
        srcv, _, _, rows, _, _, sg, _ = bufs[b]
        return [pltpu.make_async_copy(msg_hbm.at[srcv.at[j]],
                                      rows.at[pl.ds(j * CH, CH)], sg)
                for j in range(4)]

    def scatter_descs(b):
        _, dstv, _, rows, _, _, _, ss = bufs[b]
        return [pltpu.make_async_copy(rows.at[pl.ds(j * CH, CH)],
                                      shared.at[dstv.at[j]], ss)
                for j in range(4)]

    def adjust(b):
        srcv = bufs[b][0]
        for j in range(4):
            for i in range(CH // 16):
                sl = pl.ds(i * 16, 16)
                srcv[j, sl] = srcv[j, sl] + coff

    def scale(b):
        eww, rows = bufs[b][2], bufs[b][3]

        @plsc.parallel_loop(0, CB, 16, unroll=2)
        def _(i):
            wv = eww[pl.ds(i, 16)]
            for e in range(16):
                rows[i + e, :] = rows[i + e, :] * wv[e]

    def issue_scatter(b):
        _, dstv, _, rows, _, _, _, ss = bufs[b]
        for j in range(4):
            pltpu.async_copy(rows.at[pl.ds(j * CH, CH)],
                             shared.at[dstv.at[j]], ss, add=True)

    for d in srcew_descs(0, 0):
        d.start()
    dst_desc(0, 0).start()
    for d in srcew_descs(0, 0):
        d.wait()
    adjust(0)
    for d in gather_descs(0):
        d.start()
    for d in srcew_descs(1, 1):
        d.start()

    def pair(k, carry):
        for p in (0, 1):
            g = 2 * k + p
            q = 1 - p
            for d in gather_descs(p):
                d.wait()
            @pl.when(g >= 1)
            def _():
                for d in scatter_descs(q):
                    d.wait()

            @pl.when(g + 1 < NCHUNK)
            def _():
                for d in srcew_descs(q, g + 1):
                    d.wait()
                adjust(q)
                for d in gather_descs(q):
                    d.start()
                dst_desc(q, g + 1).start()
            scale(p)
            dst_desc(p, g).wait()
            issue_scatter(p)
            @pl.when(g + 2 < NCHUNK)
            def _():
                for d in srcew_descs(p, g + 2):
                    d.start()
        return carry

    lax.fori_loop(0, NCHUNK // 2, pair, 0)
    for d in scatter_descs((NCHUNK - 1) % 2):
        d.wait()
    plsc.subcore_barrier()
    pltpu.sync_copy(shared.at[pl.ds(s * ZS, ZS)],
                    out_hbm.at[c, pl.ds(s * ZS, ZS)])


@functools.partial(
    pl.kernel,
    out_type=jax.ShapeDtypeStruct((NP_PAD, HID), jnp.float32),
    mesh=_sc_mesh(),
    compiler_params=pltpu.CompilerParams(use_tc_tiling_on_sc=False),
    scratch_types=[
        pltpu.VMEM((CH,), jnp.int32),
        pltpu.VMEM((CH, HID), jnp.float32),
        pltpu.SemaphoreType.DMA,
    ],
)
def _gather_kernel(x_hbm, idx_hbm, out_hbm, idxv, rows, sem):
    c = lax.axis_index("c")
    s = lax.axis_index("s")
    wid = s * 2 + c
    for j in range(NP_PAD // (32 * CH)):
        b = wid * (NP_PAD // 32) + j * CH
        pltpu.sync_copy(idx_hbm.at[pl.ds(b, CH)], idxv)
        pltpu.async_copy(x_hbm.at[idxv], rows, sem).wait()
        pltpu.sync_copy(rows, out_hbm.at[pl.ds(b, CH)])


def kernel(node_features, edges, edge_weights, input_node_indices, params):
    p = params
    f32 = jnp.float32
    nf = node_features.reshape(N, 128)
    pad_e = E_PAD - E
    srcp = jnp.pad(edges[1].astype(jnp.int32), (0, pad_e))
    dstp = jnp.pad(edges[0].astype(jnp.int32), (0, pad_e))
    ewp = jnp.pad(edge_weights, (0, pad_e))
    idxp = jnp.pad(input_node_indices.astype(jnp.int32), (0, NP_PAD - NPRED))
    zeros_h = jnp.zeros((N_ACC, 16), f32)

    def _msg_pad(m):
        return m.reshape(2 * N_ACC, 16)

    wc, bc = _conv_mats(p['conv_k'], p['conv_b'])
    pre = _fold_ffn(p, 'pre')
    c1p = _fold_ffn(p, 'c1_p')
    c1u = _fold_ffn(p, 'c1_u')
    c2p = _fold_ffn(p, 'c2_p')
    c2u = _fold_ffn(p, 'c2_u')
    post = _fold_ffn(p, 'post')

    def split_u(u):
        w1, b1, w2, b2 = u
        return (w1[:HID], w1[HID:HID + 16], w1[HID + 16:], b1, w2, b2)

    c1u = split_u(c1u)
    c2u = split_u(c2u)

    ew2 = ewp.reshape(E_PAD // CH, CH)
    ssum = pl.pallas_call(
        _sum_body,
        grid=(8,),
        in_specs=[pl.BlockSpec((E_PAD // CH // 8, 128), lambda i: (i, 0))],
        out_specs=pl.BlockSpec(memory_space=pltpu.SMEM),
        out_shape=jax.ShapeDtypeStruct((1, 1), f32),
    )(ew2)

    wspecs = [_full(w.shape) for w in (wc, bc, *pre, *c1p)]
    x0, msg1 = pl.pallas_call(
        _encode_body,
        grid=(NBLK,),
        in_specs=[pl.BlockSpec((BT, 128), lambda i: (i, 0))] + wspecs,
        out_specs=[pl.BlockSpec((BT, HID), lambda i: (i, 0)),
                   pl.BlockSpec((2, BT, 16), lambda i: (0, i, 0))],
        out_shape=[jax.ShapeDtypeStruct((N, HID), f32),
                   jax.ShapeDtypeStruct((2, N_ACC, 16), f32)],
    )(nf, wc, bc, *pre, *c1p)

    src2d = srcp.reshape(E_PAD // CH, CH)
    dst2d = dstp.reshape(E_PAD // CH, CH)
    agg1 = _agg_kernel(_msg_pad(msg1), src2d, dst2d, ewp, zeros_h)

    def update(x, agg, uw, mw, body, n_out):
        ins = (x, agg, ssum, *uw, *mw)
        specs = [pl.BlockSpec((BT, HID), lambda i: (i, 0)),
                 pl.BlockSpec((2, BT, 16), lambda i: (0, i, 0)),
                 pl.BlockSpec(memory_space=pltpu.SMEM)]
        specs += [_full(w.shape) for w in (*uw, *mw)]
        if n_out == 2:
            outs = [pl.BlockSpec((BT, HID), lambda i: (i, 0)),
                    pl.BlockSpec((2, BT, 16), lambda i: (0, i, 0))]
            oshape = [jax.ShapeDtypeStruct((N, HID), f32),
                      jax.ShapeDtypeStruct((2, N_ACC, 16), f32)]
        else:
            outs = pl.BlockSpec((BT, HID), lambda i: (i, 0))
            oshape = jax.ShapeDtypeStruct((N, HID), f32)
        return pl.pallas_call(body, grid=(NBLK,), in_specs=specs,
                              out_specs=outs, out_shape=oshape)(*ins)

    x1, msg2 = update(x0, agg1, c1u, c2p, _update_body, 2)
    agg2 = _agg_kernel(_msg_pad(msg2), src2d, dst2d, ewp, zeros_h)
    x3 = update(x1, agg2, c2u, post, _final_body, 1)

    emb = _gather_kernel(x3, idxp)

    logits = pl.pallas_call(
        _logits_body,
        grid=(NP_PAD // 1024,),
        in_specs=[pl.BlockSpec((1024, HID), lambda i: (i, 0)),
                  _full((HID, NCLS)), _full((1, NCLS))],
        out_specs=pl.BlockSpec((1024, NCLS), lambda i: (i, 0)),
        out_shape=jax.ShapeDtypeStruct((NP_PAD, NCLS), f32),
    )(emb, p['log_w'], p['log_b'][None, :])

    return logits[:NPRED]


# speedup vs baseline: 12.9627x; 1.1094x over previous
import functools
import numpy as np
import jax
import jax.numpy as jnp
from jax import lax
from jax.experimental import pallas as pl
from jax.experimental.pallas import tpu as pltpu
from jax.experimental.pallas import tpu_sc as plsc

N = 100000
E = 1600000
NPRED = 10000
NCLS = 40
HID = 32
BN_EPS = 1e-3

NSUB = 16
CH = 128
CB = 512
NSTR = CB // CH
E_PAD = 1605632
EPS_E = E_PAD // NSUB
NCHUNK = EPS_E // CB
N_ACC = 100096
ZS = N_ACC // NSUB
NP_PAD = 12288

BT = 2000
NBLK = N // BT


def _gelu(x):
    return x * 0.5 * (1.0 + lax.erf(x * np.float32(1.0 / np.sqrt(2.0))))


def _fold_ffn(p, pre):
    s1 = p[pre + '_bn1_g'] * np.float32(1.0 / np.sqrt(1.0 + BN_EPS))
    w1 = s1[:, None] * p[pre + '_d1_w']
    b1 = p[pre + '_bn1_b'] @ p[pre + '_d1_w'] + p[pre + '_d1_b']
    s2 = p[pre + '_bn2_g'] * np.float32(1.0 / np.sqrt(1.0 + BN_EPS))
    w2 = s2[:, None] * p[pre + '_d2_w']
    b2 = p[pre + '_bn2_b'] @ p[pre + '_d2_w'] + p[pre + '_d2_b']
    return w1, b1[None, :], w2, b2[None, :]


def _conv_mats(k, b):
    sel = np.zeros((4 * 128 * 32, 9), np.float32)
    for qi in range(2):
        for qj in range(2):
            q = qi * 2 + qj
            for a in range(3):
                for bb in range(3):
                    for pi in range(8):
                        for pj in range(4):
                            ii = 2 * pi + qi + a - 1
                            jj = 2 * pj + qj + bb - 1
                            if 0 <= ii < 16 and 0 <= jj < 8:
                                sel[(q * 128 + ii * 8 + jj) * 32 + pi * 4 + pj,
                                    a * 3 + bb] = 1.0
    wc = (jnp.asarray(sel) @ k.reshape(9, 4)).reshape(4, 128, 128)
    bvec = jnp.tile(b, 32)[None, :]
    return wc, bvec


def _full(spec_shape):
    return pl.BlockSpec(spec_shape, lambda i: tuple(0 for _ in spec_shape))


def _sum_body(ew_ref, out_ref):
    @pl.when(pl.program_id(0) == 0)
    def _():
        out_ref[0, 0] = 0.0
    out_ref[0, 0] += jnp.sum(ew_ref[...])


def _encode_body(nf_ref, wc_ref, bc_ref, pw1, pb1, pw2, pb2,
                 mw1, mb1, mw2, mb2, x0_ref, msg_ref):
    nf = nf_ref[...]
    y = jnp.maximum(jnp.dot(nf, wc_ref[0], preferred_element_type=jnp.float32)
                    + bc_ref[...], 0.0)
    for q in range(1, 4):
        t = jnp.maximum(jnp.dot(nf, wc_ref[q], preferred_element_type=jnp.float32)
                        + bc_ref[...], 0.0)
        y = jnp.maximum(y, t)
    x = _gelu(jnp.dot(y, pw1[...], preferred_element_type=jnp.float32) + pb1[...])
    x = _gelu(jnp.dot(x, pw2[...], preferred_element_type=jnp.float32) + pb2[...])
    m = _gelu(jnp.dot(x, mw1[...], preferred_element_type=jnp.float32) + mb1[...])
    m = _gelu(jnp.dot(m, mw2[...], preferred_element_type=jnp.float32) + mb2[...])
    x0_ref[...] = x
    msg_ref[0] = m[:, :16]
    msg_ref[1] = m[:, 16:]


def _update_body(x_ref, agg_ref, s_ref, uw1a, uw1b0, uw1b1, ub1, uw2, ub2,
                 mw1, mb1, mw2, mb2, x1_ref, msg_ref):
    inv = 1.0 / s_ref[0, 0]
    x = x_ref[...]
    ga = jnp.dot(agg_ref[0], uw1b0[...], preferred_element_type=jnp.float32)
    gb = jnp.dot(agg_ref[1], uw1b1[...], preferred_element_type=jnp.float32)
    u = _gelu(jnp.dot(x, uw1a[...], preferred_element_type=jnp.float32)
              + (ga + gb) * inv + ub1[...])
    u = _gelu(jnp.dot(u, uw2[...], preferred_element_type=jnp.float32) + ub2[...])
    u = u * lax.rsqrt(jnp.maximum(jnp.sum(u * u, axis=1, keepdims=True), 1e-12))
    x1 = u + x
    m = _gelu(jnp.dot(x1, mw1[...], preferred_element_type=jnp.float32) + mb1[...])
    m = _gelu(jnp.dot(m, mw2[...], preferred_element_type=jnp.float32) + mb2[...])
    x1_ref[...] = x1
    msg_ref[0] = m[:, :16]
    msg_ref[1] = m[:, 16:]


def _final_body(x_ref, agg_ref, s_ref, uw1a, uw1b0, uw1b1, ub1, uw2, ub2,
                pw1, pb1, pw2, pb2, x3_ref):
    inv = 1.0 / s_ref[0, 0]
    x = x_ref[...]
    ga = jnp.dot(agg_ref[0], uw1b0[...], preferred_element_type=jnp.float32)
    gb = jnp.dot(agg_ref[1], uw1b1[...], preferred_element_type=jnp.float32)
    u = _gelu(jnp.dot(x, uw1a[...], preferred_element_type=jnp.float32)
              + (ga + gb) * inv + ub1[...])
    u = _gelu(jnp.dot(u, uw2[...], preferred_element_type=jnp.float32) + ub2[...])
    u = u * lax.rsqrt(jnp.maximum(jnp.sum(u * u, axis=1, keepdims=True), 1e-12))
    x2 = u + x
    t = _gelu(jnp.dot(x2, pw1[...], preferred_element_type=jnp.float32) + pb1[...])
    x3_ref[...] = _gelu(jnp.dot(t, pw2[...], preferred_element_type=jnp.float32)
                        + pb2[...])


def _logits_body(emb_ref, w_ref, b_ref, out_ref):
    out_ref[...] = (jnp.dot(emb_ref[...], w_ref[...],
                            preferred_element_type=jnp.float32) + b_ref[...])


def _sc_mesh():
    return plsc.VectorSubcoreMesh(core_axis_name="c", subcore_axis_name="s",
                                  num_cores=2, num_subcores=NSUB)


@functools.partial(
    pl.kernel,
    out_type=jax.ShapeDtypeStruct((2, N_ACC, 16), jnp.float32),
    mesh=_sc_mesh(),
    compiler_params=pltpu.CompilerParams(use_tc_tiling_on_sc=False),
    scratch_types=[
        pltpu.VMEM((NSTR, CH), jnp.int32),
        pltpu.VMEM((NSTR, CH), jnp.int32),
        pltpu.VMEM((NSTR, CH), jnp.int32),
        pltpu.VMEM((NSTR, CH), jnp.int32),
        pltpu.VMEM((CB,), jnp.float32),
        pltpu.VMEM((CB,), jnp.float32),
        pltpu.VMEM((CB, 16), jnp.float32),
        pltpu.VMEM((CB, 16), jnp.float32),
        pltpu.VMEM_SHARED((N_ACC, 16), jnp.float32),
        pltpu.SemaphoreType.DMA,
        pltpu.SemaphoreType.DMA,
        pltpu.SemaphoreType.DMA,
        pltpu.SemaphoreType.DMA,
        pltpu.SemaphoreType.DMA,
        pltpu.SemaphoreType.DMA,
        pltpu.SemaphoreType.DMA,
        pltpu.SemaphoreType.DMA,
    ],
)
def _agg_kernel(msg_hbm, src_hbm, dst_hbm, ew_hbm, zero_hbm, out_hbm,
                srcv0, srcv1, dstv0, dstv1, eww0, eww1, rows0, rows1,
                shared, si0, si1, sd0, sd1, sg0, sg1, ss0, ss1):
    c = lax.axis_index("c")
    s = lax.axis_index("s")
    pltpu.sync_copy(zero_hbm.at[pl.ds(s * ZS, ZS)], shared.at[pl.ds(s * ZS, ZS)])
    plsc.subcore_barrier()
    ebase = s * EPS_E
    rbase = s * (EPS_E // CH)
    coff = c * N_ACC

    bufs = ((srcv0, dstv0, eww0, rows0, si0, sd0, sg0, ss0),
            (srcv1, dstv1, eww1, rows1, si1, sd1, sg1, ss1))

    def srcew_descs(b, g):
        srcv, _, eww, _, si, _, _, _ = bufs[b]
        return [pltpu.make_async_copy(src_hbm.at[pl.ds(rbase + g * NSTR, NSTR)], srcv, si),
                pltpu.make_async_copy(ew_hbm.at[pl.ds(ebase + g * CB, CB)], eww, si)]

    def dst_desc(b, g):
        _, dstv, _, _, _, sd, _, _ = bufs[b]
        return pltpu.make_async_copy(dst_hbm.at[pl.ds(rbase + g * NSTR, NSTR)], dstv, sd)

    def gather_descs(b):
        srcv, _, _, rows, _, _, sg, _ = bufs[b]
        return [pltpu.make_async_copy(msg_hbm.at[srcv.at[j]],
                                      rows.at[pl.ds(j * CH, CH)], sg)
                for j in range(NSTR)]

    def scatter_descs(b):
        _, dstv, _, rows, _, _, _, ss = bufs[b]
        return [pltpu.make_async_copy(rows.at[pl.ds(j * CH, CH)],
                                      shared.at[dstv.at[j]], ss)
                for j in range(NSTR)]

    def adjust(b):
        srcv = bufs[b][0]
        for j in range(NSTR):
            for i in range(CH // 16):
                sl = pl.ds(i * 16, 16)
                srcv[j, sl] = srcv[j, sl] + coff

    def scale(b):
        eww, rows = bufs[b][2], bufs[b][3]

        @plsc.parallel_loop(0, CB, 16, unroll=2)
        def _(i):
            wv = eww[pl.ds(i, 16)]
            for e in range(16):
                rows[i + e, :] = rows[i + e, :] * wv[e]

    def issue_scatter(b):
        _, dstv, _, rows, _, _, _, ss = bufs[b]
        for j in range(NSTR):
            pltpu.async_copy(rows.at[pl.ds(j * CH, CH)],
                             shared.at[dstv.at[j]], ss, add=True)

    for d in srcew_descs(0, 0):
        d.start()
    dst_desc(0, 0).start()
    for d in srcew_descs(0, 0):
        d.wait()
    adjust(0)
    for d in gather_descs(0):
        d.start()
    for d in srcew_descs(1, 1):
        d.start()

    def pair(k, carry):
        for p in (0, 1):
            g = 2 * k + p
            q = 1 - p
            for d in gather_descs(p):
                d.wait()
            @pl.when(g >= 1)
            def _():
                for d in scatter_descs(q):
                    d.wait()

            @pl.when(g + 1 < NCHUNK)
            def _():
                for d in srcew_descs(q, g + 1):
                    d.wait()
                adjust(q)
                for d in gather_descs(q):
                    d.start()
                dst_desc(q, g + 1).start()
            scale(p)
            dst_desc(p, g).wait()
            issue_scatter(p)
            @pl.when(g + 2 < NCHUNK)
            def _():
                for d in srcew_descs(p, g + 2):
                    d.start()
        return carry

    lax.fori_loop(0, NCHUNK // 2, pair, 0)
    for d in scatter_descs((NCHUNK - 1) % 2):
        d.wait()
    plsc.subcore_barrier()
    pltpu.sync_copy(shared.at[pl.ds(s * ZS, ZS)],
                    out_hbm.at[c, pl.ds(s * ZS, ZS)])


@functools.partial(
    pl.kernel,
    out_type=jax.ShapeDtypeStruct((NP_PAD, HID), jnp.float32),
    mesh=_sc_mesh(),
    compiler_params=pltpu.CompilerParams(use_tc_tiling_on_sc=False),
    scratch_types=[
        pltpu.VMEM((CH,), jnp.int32),
        pltpu.VMEM((CH, HID), jnp.float32),
        pltpu.SemaphoreType.DMA,
    ],
)
def _gather_kernel(x_hbm, idx_hbm, out_hbm, idxv, rows, sem):
    c = lax.axis_index("c")
    s = lax.axis_index("s")
    wid = s * 2 + c
    for j in range(NP_PAD // (32 * CH)):
        b = wid * (NP_PAD // 32) + j * CH
        pltpu.sync_copy(idx_hbm.at[pl.ds(b, CH)], idxv)
        pltpu.async_copy(x_hbm.at[idxv], rows, sem).wait()
        pltpu.sync_copy(rows, out_hbm.at[pl.ds(b, CH)])


def kernel(node_features, edges, edge_weights, input_node_indices, params):
    p = params
    f32 = jnp.float32
    nf = node_features.reshape(N, 128)
    pad_e = E_PAD - E
    srcp = jnp.pad(edges[1].astype(jnp.int32), (0, pad_e))
    dstp = jnp.pad(edges[0].astype(jnp.int32), (0, pad_e))
    ewp = jnp.pad(edge_weights, (0, pad_e))
    idxp = jnp.pad(input_node_indices.astype(jnp.int32), (0, NP_PAD - NPRED))
    zeros_h = jnp.zeros((N_ACC, 16), f32)

    def _msg_pad(m):
        return m.reshape(2 * N_ACC, 16)

    wc, bc = _conv_mats(p['conv_k'], p['conv_b'])
    pre = _fold_ffn(p, 'pre')
    c1p = _fold_ffn(p, 'c1_p')
    c1u = _fold_ffn(p, 'c1_u')
    c2p = _fold_ffn(p, 'c2_p')
    c2u = _fold_ffn(p, 'c2_u')
    post = _fold_ffn(p, 'post')

    def split_u(u):
        w1, b1, w2, b2 = u
        return (w1[:HID], w1[HID:HID + 16], w1[HID + 16:], b1, w2, b2)

    c1u = split_u(c1u)
    c2u = split_u(c2u)

    ew2 = ewp.reshape(E_PAD // CH, CH)
    ssum = pl.pallas_call(
        _sum_body,
        grid=(8,),
        in_specs=[pl.BlockSpec((E_PAD // CH // 8, 128), lambda i: (i, 0))],
        out_specs=pl.BlockSpec(memory_space=pltpu.SMEM),
        out_shape=jax.ShapeDtypeStruct((1, 1), f32),
    )(ew2)

    wspecs = [_full(w.shape) for w in (wc, bc, *pre, *c1p)]
    x0, msg1 = pl.pallas_call(
        _encode_body,
        grid=(NBLK,),
        in_specs=[pl.BlockSpec((BT, 128), lambda i: (i, 0))] + wspecs,
        out_specs=[pl.BlockSpec((BT, HID), lambda i: (i, 0)),
                   pl.BlockSpec((2, BT, 16), lambda i: (0, i, 0))],
        out_shape=[jax.ShapeDtypeStruct((N, HID), f32),
                   jax.ShapeDtypeStruct((2, N_ACC, 16), f32)],
    )(nf, wc, bc, *pre, *c1p)

    src2d = srcp.reshape(E_PAD // CH, CH)
    dst2d = dstp.reshape(E_PAD // CH, CH)
    agg1 = _agg_kernel(_msg_pad(msg1), src2d, dst2d, ewp, zeros_h)

    def update(x, agg, uw, mw, body, n_out):
        ins = (x, agg, ssum, *uw, *mw)
        specs = [pl.BlockSpec((BT, HID), lambda i: (i, 0)),
                 pl.BlockSpec((2, BT, 16), lambda i: (0, i, 0)),
                 pl.BlockSpec(memory_space=pltpu.SMEM)]
        specs += [_full(w.shape) for w in (*uw, *mw)]
        if n_out == 2:
            outs = [pl.BlockSpec((BT, HID), lambda i: (i, 0)),
                    pl.BlockSpec((2, BT, 16), lambda i: (0, i, 0))]
            oshape = [jax.ShapeDtypeStruct((N, HID), f32),
                      jax.ShapeDtypeStruct((2, N_ACC, 16), f32)]
        else:
            outs = pl.BlockSpec((BT, HID), lambda i: (i, 0))
            oshape = jax.ShapeDtypeStruct((N, HID), f32)
        return pl.pallas_call(body, grid=(NBLK,), in_specs=specs,
                              out_specs=outs, out_shape=oshape)(*ins)

    x1, msg2 = update(x0, agg1, c1u, c2p, _update_body, 2)
    agg2 = _agg_kernel(_msg_pad(msg2), src2d, dst2d, ewp, zeros_h)
    x3 = update(x1, agg2, c2u, post, _final_body, 1)

    emb = _gather_kernel(x3, idxp)

    logits = pl.pallas_call(
        _logits_body,
        grid=(NP_PAD // 1024,),
        in_specs=[pl.BlockSpec((1024, HID), lambda i: (i, 0)),
                  _full((HID, NCLS)), _full((1, NCLS))],
        out_specs=pl.BlockSpec((1024, NCLS), lambda i: (i, 0)),
        out_shape=jax.ShapeDtypeStruct((NP_PAD, NCLS), f32),
    )(emb, p['log_w'], p['log_b'][None, :])

    return logits[:NPRED]
